# baseline (speedup 1.0000x reference)
import jax
import jax.numpy as jnp
from jax import lax
from jax.experimental import pallas as pl
from jax.experimental.pallas import tpu as pltpu, tpu_sc as plsc

VOCAB = 100000
EMBED = 64
PAD_IDX = 0
B, L = 16384, 50
N = B * L

NC, NS, LANES = 2, 16, 16
NW = NC * NS
ROWS_W = B // NW
HALF = ROWS_W // 2
CHUNKS = 2 * L
RUN = 2 * 8 * 128


def _body(vocab_hbm, uniq_hbm, shuf_hbm, table_hbm, out_hbm,
          ui_blk, vi_blk, ui_c0, ui_c1, ridx0, ridx1, mask0, mask1,
          rows0, rows1, trans0, trans1,
          l1s0, l1s1, gs0, gs1, ws0, ws1):
    wid = lax.axis_index("s") * NC + lax.axis_index("c")
    pos_w = wid * ROWS_W * L
    bhi_w = wid * (ROWS_W // 128)

    ui_c = (ui_c0, ui_c1)
    ridx = (ridx0, ridx1)
    mask = (mask0, mask1)
    rows = (rows0, rows1)
    trans = (trans0, trans1)
    l1s = (l1s0, l1s1)
    gs = (gs0, gs1)
    ws = (ws0, ws1)

    pltpu.sync_copy(uniq_hbm.at[pl.ds(pos_w, ROWS_W * L)], ui_blk)
    pltpu.sync_copy(vocab_hbm.at[pl.ds(pos_w, ROWS_W * L)], vi_blk)

    lane = lax.iota(jnp.int32, LANES)

    def stage(c, s):
        l = c >> 1
        half = c & 1
        base = half * HALF * L + l

        def grp(g, acc):
            idx = (base + g * LANES * L) + lane * L
            ui_c[s][pl.ds(g * LANES, LANES)] = plsc.load_gather(ui_blk, [idx])
            is_pad = plsc.load_gather(vi_blk, [idx]) == PAD_IDX
            mask[s][pl.ds(g * LANES, LANES)] = jnp.where(is_pad, 0.0, 1.0)
            return acc + jnp.where(is_pad, 1, 0)

        acc = lax.fori_loop(0, HALF // LANES, grp,
                            jnp.zeros((LANES,), jnp.int32))
        pltpu.async_copy(shuf_hbm.at[ui_c[s]], ridx[s], l1s[s])
        return lax.reduce_sum(acc, axes=(0,))

    def row_gather(s):
        pltpu.make_async_copy(shuf_hbm.at[ui_c[s]], ridx[s], l1s[s]).wait()
        pltpu.async_copy(table_hbm.at[ridx[s]], rows[s], gs[s])

    def finish(c, s, pads):
        pltpu.make_async_copy(table_hbm.at[ridx[s]], rows[s], gs[s]).wait()

        @pl.when(pads > 0)
        def _fixup():
            def rowfix(r, _):
                m = plsc.load_gather(mask[s],
                                     [jnp.full((LANES,), r, jnp.int32)])
                for j in range(EMBED // LANES):
                    sl = pl.ds(j * LANES, LANES)
                    rows[s][r, sl] = rows[s][r, sl] * m
                return 0
            lax.fori_loop(0, HALF, rowfix, 0)

        @plsc.parallel_loop(0, EMBED, unroll=8)
        def _t(e):
            col = jnp.full((LANES,), e, jnp.int32)
            base_e = ((e >> 3) << 11) | ((e & 7) << 7)
            for bh in range(2):
                for g in range(8):
                    rvec = lane + (bh * 128 + g * LANES)
                    trans[s][pl.ds(base_e + bh * 1024 + g * LANES, LANES)] = (
                        plsc.load_gather(rows[s], [rvec, col]))

        l = c >> 1
        half = c & 1
        for e_hi in range(8):
            off = l * (B * EMBED) + e_hi * (B * 8) + (bhi_w + half * 2) * 1024
            pltpu.async_copy(trans[s].at[pl.ds(e_hi * RUN, RUN)],
                             out_hbm.at[pl.ds(off, RUN)], ws[s])

    def wait_wb(c, s):
        l = c >> 1
        half = c & 1
        for e_hi in range(8):
            off = l * (B * EMBED) + e_hi * (B * 8) + (bhi_w + half * 2) * 1024
            pltpu.make_async_copy(trans[s].at[pl.ds(e_hi * RUN, RUN)],
                                  out_hbm.at[pl.ds(off, RUN)], ws[s]).wait()

    pads_a = stage(0, 0)
    row_gather(0)
    pads_b = stage(1, 1)

    row_gather(1)
    finish(0, 0, pads_a)
    pads_a = stage(2, 0)
    finish(1, 1, pads_b)
    wait_wb(0, 0)
    pads_b = stage(3, 1)
    row_gather(0)

    def pair(k, carry):
        pads_a, pads_b = carry
        a = 2 * k
        b = a + 1
        row_gather(1)
        finish(a, 0, pads_a)
        wait_wb(b - 2, 1)
        pads_a2 = stage(a + 2, 0)
        finish(b, 1, pads_b)
        wait_wb(a, 0)
        pads_b2 = stage(b + 2, 1)
        row_gather(0)
        return pads_a2, pads_b2

    pads_a, pads_b = lax.fori_loop(1, CHUNKS // 2 - 1, pair, (pads_a, pads_b))
    row_gather(1)
    finish(CHUNKS - 2, 0, pads_a)
    wait_wb(CHUNKS - 3, 1)
    finish(CHUNKS - 1, 1, pads_b)
    wait_wb(CHUNKS - 2, 0)
    wait_wb(CHUNKS - 1, 1)


@jax.jit
def _run(vocab_flat, uniq_flat, shuffle, table):
    mesh = plsc.VectorSubcoreMesh(core_axis_name="c", subcore_axis_name="s")
    f = pl.kernel(
        _body,
        out_type=jax.ShapeDtypeStruct((N * EMBED,), jnp.float32),
        mesh=mesh,
        compiler_params=pltpu.CompilerParams(needs_layout_passes=False,
                                             use_tc_tiling_on_sc=False),
        scratch_types=[
            pltpu.VMEM((ROWS_W * L,), jnp.int32),
            pltpu.VMEM((ROWS_W * L,), jnp.int32),
            pltpu.VMEM((HALF,), jnp.int32),
            pltpu.VMEM((HALF,), jnp.int32),
            pltpu.VMEM((HALF,), jnp.int32),
            pltpu.VMEM((HALF,), jnp.int32),
            pltpu.VMEM((HALF,), jnp.float32),
            pltpu.VMEM((HALF,), jnp.float32),
            pltpu.VMEM((HALF, EMBED), jnp.float32),
            pltpu.VMEM((HALF, EMBED), jnp.float32),
            pltpu.VMEM((8 * RUN,), jnp.float32),
            pltpu.VMEM((8 * RUN,), jnp.float32),
            pltpu.SemaphoreType.DMA,
            pltpu.SemaphoreType.DMA,
            pltpu.SemaphoreType.DMA,
            pltpu.SemaphoreType.DMA,
            pltpu.SemaphoreType.DMA,
            pltpu.SemaphoreType.DMA,
        ],
    )
    return f(vocab_flat, uniq_flat, shuffle, table)


def kernel(vocab_word_idx, batch_unique_word_idx,
           obfuscation_vocab_random_indices_shuffle,
           obfuscation_embedding_table):
    out = _run(vocab_word_idx.reshape(N),
               batch_unique_word_idx.reshape(N),
               obfuscation_vocab_random_indices_shuffle,
               obfuscation_embedding_table)
    out5 = out.reshape(L, EMBED // 8, B // 128, 8, 128)
    return out5.transpose(2, 4, 0, 1, 3).reshape(B, L, EMBED)


# speedup vs baseline: 1.1577x; 1.1577x over previous
import jax
import jax.numpy as jnp
from jax import lax
from jax.experimental import pallas as pl
from jax.experimental.pallas import tpu as pltpu, tpu_sc as plsc

VOCAB = 100000
EMBED = 64
PAD_IDX = 0
B, L = 16384, 50
N = B * L

NC, NS, LANES = 2, 16, 16
NW = NC * NS
ROWS_W = B // NW
HALF = ROWS_W // 2
CHUNKS = 2 * L
RUN = 2 * 8 * 128


def _body(vocab_hbm, uniq_hbm, shuf_hbm, table_hbm, out_hbm,
          ui_blk, vi_blk, ui_c0, ui_c1, ridx0, ridx1, mask0, mask1,
          rows0, rows1, trans0, trans1,
          l1s0, l1s1, gs0, gs1, ws0, ws1):
    wid = lax.axis_index("s") * NC + lax.axis_index("c")
    pos_w = wid * ROWS_W * L
    bhi_w = wid * (ROWS_W // 128)

    ui_c = (ui_c0, ui_c1)
    ridx = (ridx0, ridx1)
    mask = (mask0, mask1)
    rows = (rows0, rows1)
    trans = (trans0, trans1)
    l1s = (l1s0, l1s1)
    gs = (gs0, gs1)
    ws = (ws0, ws1)

    pltpu.sync_copy(uniq_hbm.at[pl.ds(pos_w, ROWS_W * L)], ui_blk)
    pltpu.sync_copy(vocab_hbm.at[pl.ds(pos_w, ROWS_W * L)], vi_blk)

    lane = lax.iota(jnp.int32, LANES)

    def stage(c, s):
        l = c >> 1
        half = c & 1
        base = half * HALF * L + l

        def grp(g, acc):
            idx = (base + g * LANES * L) + lane * L
            ui_c[s][pl.ds(g * LANES, LANES)] = plsc.load_gather(ui_blk, [idx])
            is_pad = plsc.load_gather(vi_blk, [idx]) == PAD_IDX
            mask[s][pl.ds(g * LANES, LANES)] = jnp.where(is_pad, 0.0, 1.0)
            return acc + jnp.where(is_pad, 1, 0)

        acc = lax.fori_loop(0, HALF // LANES, grp,
                            jnp.zeros((LANES,), jnp.int32))
        pltpu.async_copy(shuf_hbm.at[ui_c[s]], ridx[s], l1s[s])
        return lax.reduce_sum(acc, axes=(0,))

    def row_gather(s):
        pltpu.make_async_copy(shuf_hbm.at[ui_c[s]], ridx[s], l1s[s]).wait()
        pltpu.async_copy(table_hbm.at[ridx[s]], rows[s], gs[s])

    def finish(c, s, pads):
        pltpu.make_async_copy(table_hbm.at[ridx[s]], rows[s], gs[s]).wait()

        @pl.when(pads > 0)
        def _fixup():
            def rowfix(r, _):
                m = plsc.load_gather(mask[s],
                                     [jnp.full((LANES,), r, jnp.int32)])
                for j in range(EMBED // LANES):
                    sl = pl.ds(j * LANES, LANES)
                    rows[s][r, sl] = rows[s][r, sl] * m
                return 0
            lax.fori_loop(0, HALF, rowfix, 0)

        @plsc.parallel_loop(0, EMBED, unroll=4)
        def _t(e):
            col = jnp.full((LANES,), e, jnp.int32)
            base_e = ((e >> 3) << 11) | ((e & 7) << 7)
            for bh in range(2):
                for g in range(8):
                    rvec = lane + (bh * 128 + g * LANES)
                    trans[s][pl.ds(base_e + bh * 1024 + g * LANES, LANES)] = (
                        plsc.load_gather(rows[s], [rvec, col]))

        l = c >> 1
        half = c & 1
        for e_hi in range(8):
            off = l * (B * EMBED) + e_hi * (B * 8) + (bhi_w + half * 2) * 1024
            pltpu.async_copy(trans[s].at[pl.ds(e_hi * RUN, RUN)],
                             out_hbm.at[pl.ds(off, RUN)], ws[s])

    def wait_wb(c, s):
        l = c >> 1
        half = c & 1
        for e_hi in range(8):
            off = l * (B * EMBED) + e_hi * (B * 8) + (bhi_w + half * 2) * 1024
            pltpu.make_async_copy(trans[s].at[pl.ds(e_hi * RUN, RUN)],
                                  out_hbm.at[pl.ds(off, RUN)], ws[s]).wait()

    pads_a = stage(0, 0)
    row_gather(0)
    pads_b = stage(1, 1)

    row_gather(1)
    finish(0, 0, pads_a)
    pads_a = stage(2, 0)
    finish(1, 1, pads_b)
    wait_wb(0, 0)
    pads_b = stage(3, 1)
    row_gather(0)

    def pair(k, carry):
        pads_a, pads_b = carry
        a = 2 * k
        b = a + 1
        row_gather(1)
        finish(a, 0, pads_a)
        wait_wb(b - 2, 1)
        pads_a2 = stage(a + 2, 0)
        finish(b, 1, pads_b)
        wait_wb(a, 0)
        pads_b2 = stage(b + 2, 1)
        row_gather(0)
        return pads_a2, pads_b2

    pads_a, pads_b = lax.fori_loop(1, CHUNKS // 2 - 1, pair, (pads_a, pads_b))
    row_gather(1)
    finish(CHUNKS - 2, 0, pads_a)
    wait_wb(CHUNKS - 3, 1)
    finish(CHUNKS - 1, 1, pads_b)
    wait_wb(CHUNKS - 2, 0)
    wait_wb(CHUNKS - 1, 1)


@jax.jit
def _run(vocab_flat, uniq_flat, shuffle, table):
    mesh = plsc.VectorSubcoreMesh(core_axis_name="c", subcore_axis_name="s")
    f = pl.kernel(
        _body,
        out_type=jax.ShapeDtypeStruct((N * EMBED,), jnp.float32),
        mesh=mesh,
        compiler_params=pltpu.CompilerParams(needs_layout_passes=False,
                                             use_tc_tiling_on_sc=False),
        scratch_types=[
            pltpu.VMEM((ROWS_W * L,), jnp.int32),
            pltpu.VMEM((ROWS_W * L,), jnp.int32),
            pltpu.VMEM((HALF,), jnp.int32),
            pltpu.VMEM((HALF,), jnp.int32),
            pltpu.VMEM((HALF,), jnp.int32),
            pltpu.VMEM((HALF,), jnp.int32),
            pltpu.VMEM((HALF,), jnp.float32),
            pltpu.VMEM((HALF,), jnp.float32),
            pltpu.VMEM((HALF, EMBED), jnp.float32),
            pltpu.VMEM((HALF, EMBED), jnp.float32),
            pltpu.VMEM((8 * RUN,), jnp.float32),
            pltpu.VMEM((8 * RUN,), jnp.float32),
            pltpu.SemaphoreType.DMA,
            pltpu.SemaphoreType.DMA,
            pltpu.SemaphoreType.DMA,
            pltpu.SemaphoreType.DMA,
            pltpu.SemaphoreType.DMA,
            pltpu.SemaphoreType.DMA,
        ],
    )
    return f(vocab_flat, uniq_flat, shuffle, table)


def kernel(vocab_word_idx, batch_unique_word_idx,
           obfuscation_vocab_random_indices_shuffle,
           obfuscation_embedding_table):
    out = _run(vocab_word_idx.reshape(N),
               batch_unique_word_idx.reshape(N),
               obfuscation_vocab_random_indices_shuffle,
               obfuscation_embedding_table)
    out5 = out.reshape(L, EMBED // 8, B // 128, 8, 128)
    return out5.transpose(2, 4, 0, 1, 3).reshape(B, L, EMBED)


# speedup vs baseline: 1.1877x; 1.0259x over previous
import jax
import jax.numpy as jnp
from jax import lax
from jax.experimental import pallas as pl
from jax.experimental.pallas import tpu as pltpu, tpu_sc as plsc

VOCAB = 100000
EMBED = 64
PAD_IDX = 0
B, L = 16384, 50
N = B * L

NC, NS, LANES = 2, 16, 16
NW = NC * NS
PER_W = N // NW
CHUNK = 160
ITERS = PER_W // CHUNK
GROUPS = CHUNK // LANES


def _body(vocab_hbm, uniq_hbm, shuf_hbm, table_hbm, out_hbm,
          shuf_v, ui_v0, ui_v1, vi_v0, vi_v1, ridx_v0, ridx_v1,
          mask_v0, mask_v1, rows_v0, rows_v1,
          gsem0, gsem1, isem0, isem1, wsem0, wsem1):
    wid = lax.axis_index("s") * NC + lax.axis_index("c")
    base_w = wid * PER_W

    ui_v = (ui_v0, ui_v1)
    vi_v = (vi_v0, vi_v1)
    ridx_v = (ridx_v0, ridx_v1)
    mask_v = (mask_v0, mask_v1)
    rows_v = (rows_v0, rows_v1)
    gsem = (gsem0, gsem1)
    isem = (isem0, isem1)
    wsem = (wsem0, wsem1)

    pltpu.sync_copy(shuf_hbm, shuf_v)

    def prefetch_idx(i, s):
        base = base_w + i * CHUNK
        pltpu.async_copy(uniq_hbm.at[pl.ds(base, CHUNK)], ui_v[s], isem[s])
        pltpu.async_copy(vocab_hbm.at[pl.ds(base, CHUNK)], vi_v[s], isem[s])

    def compute(i, s):
        base = base_w + i * CHUNK
        pltpu.make_async_copy(uniq_hbm.at[pl.ds(base, CHUNK)], ui_v[s],
                              isem[s]).wait()
        pltpu.make_async_copy(vocab_hbm.at[pl.ds(base, CHUNK)], vi_v[s],
                              isem[s]).wait()

        def grp(g, acc):
            u = ui_v[s][pl.ds(g * LANES, LANES)]
            ridx_v[s][pl.ds(g * LANES, LANES)] = plsc.load_gather(shuf_v, [u])
            is_pad = vi_v[s][pl.ds(g * LANES, LANES)] == PAD_IDX
            mask_v[s][pl.ds(g * LANES, LANES)] = jnp.where(is_pad, 0.0, 1.0)
            return acc + jnp.where(is_pad, 1, 0)

        acc = lax.fori_loop(0, GROUPS, grp, jnp.zeros((LANES,), jnp.int32))
        return lax.reduce_sum(acc, axes=(0,))

    def start_gather(s):
        pltpu.async_copy(table_hbm.at[ridx_v[s]], rows_v[s], gsem[s])

    def wait_gather(s):
        pltpu.make_async_copy(table_hbm.at[ridx_v[s]], rows_v[s],
                              gsem[s]).wait()

    def fixup_and_wb(i, s, pads):
        @pl.when(pads > 0)
        def _fixup():
            def rowfix(r, _):
                m = plsc.load_gather(mask_v[s],
                                     [jnp.full((LANES,), r, jnp.int32)])
                for j in range(EMBED // LANES):
                    sl = pl.ds(j * LANES, LANES)
                    rows_v[s][r, sl] = rows_v[s][r, sl] * m
                return 0
            lax.fori_loop(0, CHUNK, rowfix, 0)

        base = base_w + i * CHUNK
        pltpu.async_copy(rows_v[s], out_hbm.at[pl.ds(base, CHUNK)], wsem[s])

    def wait_wb(i, s):
        base = base_w + i * CHUNK
        pltpu.make_async_copy(rows_v[s], out_hbm.at[pl.ds(base, CHUNK)],
                              wsem[s]).wait()

    prefetch_idx(0, 0)
    pads_a = compute(0, 0)
    start_gather(0)
    prefetch_idx(1, 1)

    pads_b = compute(1, 1)
    prefetch_idx(2, 0)
    wait_gather(0)
    fixup_and_wb(0, 0, pads_a)
    start_gather(1)
    pads_a = compute(2, 0)
    prefetch_idx(3, 1)
    wait_gather(1)
    fixup_and_wb(1, 1, pads_b)
    wait_wb(0, 0)
    start_gather(0)

    def pair(k, pads_a):
        a = 2 * k
        b = a + 1
        pads_b = compute(b, 1)
        prefetch_idx(a + 2, 0)
        wait_gather(0)
        fixup_and_wb(a, 0, pads_a)
        wait_wb(b - 2, 1)
        start_gather(1)
        pads_a2 = compute(a + 2, 0)
        prefetch_idx(b + 2, 1)
        wait_gather(1)
        fixup_and_wb(b, 1, pads_b)
        wait_wb(a, 0)
        start_gather(0)
        return pads_a2

    pads_a = lax.fori_loop(1, ITERS // 2 - 1, pair, pads_a)
    last = ITERS - 1
    pads_b = compute(last, 1)
    wait_gather(0)
    fixup_and_wb(last - 1, 0, pads_a)
    wait_wb(last - 2, 1)
    start_gather(1)
    wait_gather(1)
    fixup_and_wb(last, 1, pads_b)
    wait_wb(last - 1, 0)
    wait_wb(last, 1)


@jax.jit
def _run(vocab_flat, uniq_flat, shuffle, table):
    mesh = plsc.VectorSubcoreMesh(core_axis_name="c", subcore_axis_name="s")
    f = pl.kernel(
        _body,
        out_type=jax.ShapeDtypeStruct((N, EMBED), jnp.float32),
        mesh=mesh,
        compiler_params=pltpu.CompilerParams(needs_layout_passes=False,
                                             use_tc_tiling_on_sc=False),
        scratch_types=[
            pltpu.VMEM((VOCAB,), jnp.int32),
            pltpu.VMEM((CHUNK,), jnp.int32),
            pltpu.VMEM((CHUNK,), jnp.int32),
            pltpu.VMEM((CHUNK,), jnp.int32),
            pltpu.VMEM((CHUNK,), jnp.int32),
            pltpu.VMEM((CHUNK,), jnp.int32),
            pltpu.VMEM((CHUNK,), jnp.int32),
            pltpu.VMEM((CHUNK,), jnp.float32),
            pltpu.VMEM((CHUNK,), jnp.float32),
            pltpu.VMEM((CHUNK, EMBED), jnp.float32),
            pltpu.VMEM((CHUNK, EMBED), jnp.float32),
            pltpu.SemaphoreType.DMA,
            pltpu.SemaphoreType.DMA,
            pltpu.SemaphoreType.DMA,
            pltpu.SemaphoreType.DMA,
            pltpu.SemaphoreType.DMA,
            pltpu.SemaphoreType.DMA,
        ],
    )
    return f(vocab_flat, uniq_flat, shuffle, table)


def kernel(vocab_word_idx, batch_unique_word_idx,
           obfuscation_vocab_random_indices_shuffle,
           obfuscation_embedding_table):
    out = _run(vocab_word_idx.reshape(N),
               batch_unique_word_idx.reshape(N),
               obfuscation_vocab_random_indices_shuffle,
               obfuscation_embedding_table)
    return out.reshape(B, L, EMBED)
